# baseline (speedup 1.0000x reference)
import jax
import jax.numpy as jnp
from jax.experimental import pallas as pl
from jax.experimental.pallas import tpu as pltpu

N = 10000
D_IN = 128
D_OUT = 128
BB = 600
_STARTS = [i * BB for i in range(16)] + [9600]
_SIZES = [BB] * 16 + [400]
NC = len(_STARTS)


def _adj_copy(adj_ref, buf, start, size, sem):
    return pltpu.make_async_copy(
        adj_ref.at[pl.ds(start, size), :], buf.at[pl.ds(0, size), :], sem
    )


def _out_copy(ostg, out_ref, start, size, sem):
    return pltpu.make_async_copy(
        ostg.at[pl.ds(0, size), :], out_ref.at[pl.ds(start, size), :], sem
    )


def _gcn_kernel(x_ref, w_ref, adj_ref, out_ref,
                xv_ref, support_ref, bufa_ref, bufb_ref,
                ostga_ref, ostgb_ref, x_sem, in_sems, out_sems):
    bufs = (bufa_ref, bufb_ref)
    ostgs = (ostga_ref, ostgb_ref)

    x_copy = pltpu.make_async_copy(x_ref, xv_ref, x_sem)
    x_copy.start()
    for c in range(2):
        _adj_copy(adj_ref, bufs[c], _STARTS[c], _SIZES[c], in_sems.at[c]).start()
    x_copy.wait()
    support_ref[...] = jnp.dot(
        xv_ref[...], w_ref[...], preferred_element_type=jnp.float32
    )

    for c in range(NC):
        slot = c % 2
        buf, ostg = bufs[slot], ostgs[slot]
        start, size = _STARTS[c], _SIZES[c]
        _adj_copy(adj_ref, buf, start, size, in_sems.at[slot]).wait()
        result = jnp.maximum(
            jnp.dot(
                buf[pl.ds(0, size), :],
                support_ref[...],
                preferred_element_type=jnp.float32,
            ),
            0.0,
        )
        if c >= 2:
            _out_copy(ostg, out_ref, _STARTS[c - 2], _SIZES[c - 2],
                      out_sems.at[slot]).wait()
        ostg[pl.ds(0, size), :] = result
        _out_copy(ostg, out_ref, start, size, out_sems.at[slot]).start()
        if c + 2 < NC:
            _adj_copy(adj_ref, buf, _STARTS[c + 2], _SIZES[c + 2],
                      in_sems.at[slot]).start()

    for c in range(NC - 2, NC):
        _out_copy(ostgs[c % 2], out_ref, _STARTS[c], _SIZES[c],
                  out_sems.at[c % 2]).wait()


def kernel(input, adj, weight):
    return pl.pallas_call(
        _gcn_kernel,
        in_specs=[
            pl.BlockSpec(memory_space=pltpu.MemorySpace.HBM),
            pl.BlockSpec((D_IN, D_OUT), lambda: (0, 0)),
            pl.BlockSpec(memory_space=pltpu.MemorySpace.HBM),
        ],
        out_specs=pl.BlockSpec(memory_space=pltpu.MemorySpace.HBM),
        out_shape=jax.ShapeDtypeStruct((N, D_OUT), jnp.float32),
        scratch_shapes=[
            pltpu.VMEM((N, D_IN), jnp.float32),
            pltpu.VMEM((N, D_OUT), jnp.float32),
            pltpu.VMEM((BB, N), jnp.float32),
            pltpu.VMEM((BB, N), jnp.float32),
            pltpu.VMEM((BB, D_OUT), jnp.float32),
            pltpu.VMEM((BB, D_OUT), jnp.float32),
            pltpu.SemaphoreType.DMA,
            pltpu.SemaphoreType.DMA((2,)),
            pltpu.SemaphoreType.DMA((2,)),
        ],
    )(input, weight, adj)


# speedup vs baseline: 1.0475x; 1.0475x over previous
import jax
import jax.numpy as jnp
from jax.experimental import pallas as pl
from jax.experimental.pallas import tpu as pltpu

N = 10000
D_IN = 128
D_OUT = 128
BM = 496


def _gcn_kernel(x_ref, w_ref, adj_ref, out_ref, support_ref):
    @pl.when(pl.program_id(0) == 0)
    def _():
        support_ref[...] = jnp.dot(
            x_ref[...], w_ref[...], preferred_element_type=jnp.float32
        )

    acc = jnp.dot(
        adj_ref[...], support_ref[...], preferred_element_type=jnp.float32
    )
    out_ref[...] = jnp.maximum(acc, 0.0)


def kernel(input, adj, weight):
    grid = (pl.cdiv(N, BM),)
    return pl.pallas_call(
        _gcn_kernel,
        grid=grid,
        in_specs=[
            pl.BlockSpec((N, D_IN), lambda i: (0, 0)),
            pl.BlockSpec((D_IN, D_OUT), lambda i: (0, 0)),
            pl.BlockSpec((BM, N), lambda i: (i, 0)),
        ],
        out_specs=pl.BlockSpec((BM, D_OUT), lambda i: (i, 0)),
        out_shape=jax.ShapeDtypeStruct((N, D_OUT), jnp.float32),
        scratch_shapes=[pltpu.VMEM((N, D_OUT), jnp.float32)],
        compiler_params=pltpu.CompilerParams(
            dimension_semantics=("arbitrary",),
        ),
    )(input, weight, adj)


# speedup vs baseline: 1.0704x; 1.0218x over previous
import jax
import jax.numpy as jnp
from jax.experimental import pallas as pl
from jax.experimental.pallas import tpu as pltpu

N = 10000
D_IN = 128
D_OUT = 128
BM = 400


def _gcn_kernel(x_ref, w_ref, adj_ref, out_ref, support_ref):
    @pl.when(pl.program_id(0) == 0)
    def _():
        support_ref[...] = jnp.dot(
            x_ref[...], w_ref[...], preferred_element_type=jnp.float32
        )

    acc = jnp.dot(
        adj_ref[...], support_ref[...], preferred_element_type=jnp.float32
    )
    out_ref[...] = jnp.maximum(acc, 0.0)


def kernel(input, adj, weight):
    grid = (pl.cdiv(N, BM),)
    return pl.pallas_call(
        _gcn_kernel,
        grid=grid,
        in_specs=[
            pl.BlockSpec((N, D_IN), lambda i: (0, 0)),
            pl.BlockSpec((D_IN, D_OUT), lambda i: (0, 0)),
            pl.BlockSpec((BM, N), lambda i: (i, 0)),
        ],
        out_specs=pl.BlockSpec((BM, D_OUT), lambda i: (i, 0)),
        out_shape=jax.ShapeDtypeStruct((N, D_OUT), jnp.float32),
        scratch_shapes=[pltpu.VMEM((N, D_OUT), jnp.float32)],
        compiler_params=pltpu.CompilerParams(
            dimension_semantics=("arbitrary",),
        ),
    )(input, weight, adj)
